# baseline (speedup 1.0000x reference)
import functools

import jax
import jax.numpy as jnp
from jax import lax
from jax.experimental import pallas as pl
from jax.experimental.pallas import tpu as pltpu
from jax.experimental.pallas import tpu_sc as plsc

N = 10000
E = 320000
D = 128
G = 64

NC = 2
NS = 16
NW = NC * NS
EW = E // NW
K = 80
NCH = EW // K
R0 = 624

_mesh = functools.partial(
    plsc.VectorSubcoreMesh, core_axis_name="c", subcore_axis_name="s")


def _edge_layer_call(xl, xr, src, dst, att):
  NB2 = 2
  NB4 = 4

  @functools.partial(
      pl.kernel,
      out_type=(jax.ShapeDtypeStruct((NC, N, D), jnp.float32),
                jax.ShapeDtypeStruct((NC, 1, N), jnp.float32)),
      mesh=_mesh(),
      compiler_params=pltpu.CompilerParams(needs_layout_passes=False),
      scratch_types=[
          [pltpu.VMEM((K,), jnp.int32)] * NB4,
          [pltpu.VMEM((K,), jnp.int32)] * NB4,
          [pltpu.VMEM((K, D), jnp.float32)] * NB2,
          [pltpu.VMEM((K, D), jnp.float32)] * NB2,
          [pltpu.VMEM((K + 16,), jnp.float32)] * NB2,
          pltpu.VMEM((D,), jnp.float32),
          pltpu.VMEM((256,), jnp.float32),
          pltpu.VMEM((624,), jnp.float32),
          pltpu.VMEM((8, D), jnp.float32),
          pltpu.VMEM_SHARED((N, D), jnp.float32),
          pltpu.VMEM_SHARED((N,), jnp.float32),
          [pltpu.SemaphoreType.DMA] * NB4,
          [pltpu.SemaphoreType.DMA] * NB2,
          [pltpu.SemaphoreType.DMA] * NB2,
          [pltpu.SemaphoreType.DMA] * NB2,
          [pltpu.SemaphoreType.DMA] * NB2,
      ])
  def k(xl_hbm, xr_hbm, src_hbm, dst_hbm, att_hbm, num_hbm, den_hbm,
        src_v, dst_v, gl_v, gr_v, ex_v, att_v, m_v, zden_v,
        zrow_v, acc_sh, denom_sh, sem_i, sem_g, sem_r, sem_sr, sem_se):
    cid = lax.axis_index("c")
    sid = lax.axis_index("s")
    wid = cid * NS + sid

    def zd(i, _):
      zden_v[pl.ds(i * 16, 16)] = jnp.zeros((16,), jnp.float32)
      return 0
    lax.fori_loop(0, 624 // 16, zd, 0)

    for r in range(8):
      for g in range(D // 16):
        zrow_v[r, pl.ds(g * 16, 16)] = jnp.zeros((16,), jnp.float32)

    pltpu.sync_copy(zden_v, denom_sh.at[pl.ds(sid * R0, R0)])

    def zc(i, _):
      pltpu.sync_copy(zrow_v, acc_sh.at[pl.ds(sid * R0 + i * 8, 8)])
      return 0
    lax.fori_loop(0, R0 // 8, zc, 0)

    @pl.when(sid == 0)
    def _():
      pltpu.sync_copy(zden_v.at[pl.ds(0, 16)],
                      denom_sh.at[pl.ds(NS * R0, N - NS * R0)])
      pltpu.sync_copy(zrow_v, acc_sh.at[pl.ds(NS * R0, 8)])
      pltpu.sync_copy(zrow_v, acc_sh.at[pl.ds(NS * R0 + 8, 8)])

    pltpu.sync_copy(att_hbm, att_v)
    plsc.subcore_barrier()

    lane = lax.iota(jnp.int32, 16)
    lane16 = lane * 16

    def start_idx(c, bi):
      base = wid * EW + c * K
      cps = pltpu.async_copy(src_hbm.at[pl.ds(base, K)], src_v[bi],
                             sem_i[bi])
      cpd = pltpu.async_copy(dst_hbm.at[pl.ds(base, K)], dst_v[bi],
                             sem_i[bi])
      del cps, cpd

    def wait_idx(c, bi):
      base = wid * EW + c * K
      pltpu.make_async_copy(src_hbm.at[pl.ds(base, K)], src_v[bi],
                            sem_i[bi]).wait()
      pltpu.make_async_copy(dst_hbm.at[pl.ds(base, K)], dst_v[bi],
                            sem_i[bi]).wait()

    def start_gather(b, bi):
      cpl = pltpu.async_copy(xl_hbm.at[src_v[bi]], gl_v[b], sem_g[b])
      cpr = pltpu.async_copy(xr_hbm.at[dst_v[bi]], gr_v[b], sem_r[b])
      del cpl, cpr

    def wait_gather(b, bi):
      pltpu.make_async_copy(xl_hbm.at[src_v[bi]], gl_v[b], sem_g[b]).wait()
      pltpu.make_async_copy(xr_hbm.at[dst_v[bi]], gr_v[b], sem_r[b]).wait()

    def start_scatter(b, bi):
      cpa = pltpu.async_copy(gl_v[b], acc_sh.at[dst_v[bi]], sem_sr[b],
                             add=True)
      cpe = pltpu.async_copy(ex_v[b].at[pl.ds(0, K)],
                             denom_sh.at[dst_v[bi]], sem_se[b], add=True)
      del cpa, cpe

    def wait_scatter(b, bi):
      pltpu.make_async_copy(gl_v[b], acc_sh.at[dst_v[bi]],
                            sem_sr[b]).wait()
      pltpu.make_async_copy(ex_v[b].at[pl.ds(0, K)],
                            denom_sh.at[dst_v[bi]], sem_se[b]).wait()

    def compute(b):
      def jgroup(j, _):
        jb = j * 16
        att16 = [att_v[pl.ds(g * 16, 16)] for g in range(D // 16)]
        for ei in range(16):
          e = jb + ei
          acc = jnp.zeros((16,), jnp.float32)
          for g in range(D // 16):
            s = gl_v[b][e, pl.ds(g * 16, 16)] + gr_v[b][e, pl.ds(g * 16, 16)]
            s = jnp.maximum(s, 0.2 * s)
            acc = acc + s * att16[g]
          m_v[pl.ds(ei * 16, 16)] = acc
        cols = [plsc.load_gather(m_v, [lane16 + l]) for l in range(16)]
        while len(cols) > 1:
          cols = [cols[i] + cols[i + 1] for i in range(0, len(cols), 2)]
        exv = jnp.exp(cols[0])
        ex_v[b][pl.ds(jb, 16)] = exv
        for ei in range(16):
          e = jb + ei
          a = ex_v[b][pl.ds(e, 16)][0]
          for g in range(D // 16):
            gl_v[b][e, pl.ds(g * 16, 16)] = gl_v[b][e, pl.ds(g * 16, 16)] * a
        return 0

      lax.fori_loop(0, K // 16, jgroup, 0)

    start_idx(0, 0)
    start_idx(1, 1)
    wait_idx(0, 0)
    start_gather(0, 0)

    def quad(q, _):
      for s in range(NB4):
        c = q * NB4 + s
        b = s % NB2

        @pl.when(c < NCH)
        def _():
          @pl.when(c + 2 < NCH)
          def _():
            start_idx(c + 2, (s + 2) % NB4)

          @pl.when(c + 1 < NCH)
          def _():
            @pl.when(c >= 1)
            def _():
              wait_scatter((s + 1) % NB2, (s + 1) % NB4)
            wait_idx(c + 1, (s + 1) % NB4)
            start_gather((s + 1) % NB2, (s + 1) % NB4)

          wait_gather(b, s % NB4)
          compute(b)
          start_scatter(b, s % NB4)
      return 0

    lax.fori_loop(0, (NCH + NB4 - 1) // NB4, quad, 0)

    wait_scatter((NCH - 2) % NB2, (NCH - 2) % NB4)
    wait_scatter((NCH - 1) % NB2, (NCH - 1) % NB4)

    plsc.subcore_barrier()

    pltpu.sync_copy(acc_sh.at[pl.ds(sid * R0, R0)],
                    num_hbm.at[cid, pl.ds(sid * R0, R0)])

    @pl.when(sid == 0)
    def _():
      pltpu.sync_copy(acc_sh.at[pl.ds(NS * R0, N - NS * R0)],
                      num_hbm.at[cid, pl.ds(NS * R0, N - NS * R0)])
      pltpu.sync_copy(denom_sh, den_hbm.at[cid, 0])

  return k(xl, xr, src, dst, att)


def _proj_call(x, Wl, bl, Wr, br):
  BLK = 1000

  def body(x_ref, wl_ref, bl_ref, wr_ref, br_ref, xl_ref, xr_ref):
    xb = x_ref[...]
    xl_ref[...] = jnp.dot(xb, wl_ref[...],
                          preferred_element_type=jnp.float32) + bl_ref[...]
    xr_ref[...] = jnp.dot(xb, wr_ref[...],
                          preferred_element_type=jnp.float32) + br_ref[...]

  return pl.pallas_call(
      body,
      grid=(N // BLK,),
      in_specs=[
          pl.BlockSpec((BLK, D), lambda i: (i, 0)),
          pl.BlockSpec((D, D), lambda i: (0, 0)),
          pl.BlockSpec((1, D), lambda i: (0, 0)),
          pl.BlockSpec((D, D), lambda i: (0, 0)),
          pl.BlockSpec((1, D), lambda i: (0, 0)),
      ],
      out_specs=[
          pl.BlockSpec((BLK, D), lambda i: (i, 0)),
          pl.BlockSpec((BLK, D), lambda i: (i, 0)),
      ],
      out_shape=[
          jax.ShapeDtypeStruct((N, D), jnp.float32),
          jax.ShapeDtypeStruct((N, D), jnp.float32),
      ],
  )(x, Wl, bl.reshape(1, D), Wr, br.reshape(1, D))


def _combine_proj_call(nums, dens, bias, Wl, bl, Wr, br):
  BLK = 1000

  def body(p_ref, d_ref, b_ref, wl_ref, bl_ref, wr_ref, br_ref,
           xl_ref, xr_ref):
    den = d_ref[0] + d_ref[1] + 1e-16
    h = (p_ref[0] + p_ref[1]) / den + b_ref[...]
    xl_ref[...] = jnp.dot(h, wl_ref[...],
                          preferred_element_type=jnp.float32) + bl_ref[...]
    xr_ref[...] = jnp.dot(h, wr_ref[...],
                          preferred_element_type=jnp.float32) + br_ref[...]

  return pl.pallas_call(
      body,
      grid=(N // BLK,),
      in_specs=[
          pl.BlockSpec((NC, BLK, D), lambda i: (0, i, 0)),
          pl.BlockSpec((NC, BLK, 1), lambda i: (0, i, 0)),
          pl.BlockSpec((1, D), lambda i: (0, 0)),
          pl.BlockSpec((D, D), lambda i: (0, 0)),
          pl.BlockSpec((1, D), lambda i: (0, 0)),
          pl.BlockSpec((D, D), lambda i: (0, 0)),
          pl.BlockSpec((1, D), lambda i: (0, 0)),
      ],
      out_specs=[
          pl.BlockSpec((BLK, D), lambda i: (i, 0)),
          pl.BlockSpec((BLK, D), lambda i: (i, 0)),
      ],
      out_shape=[
          jax.ShapeDtypeStruct((N, D), jnp.float32),
          jax.ShapeDtypeStruct((N, D), jnp.float32),
      ],
  )(nums, dens, bias.reshape(1, D), Wl, bl.reshape(1, D),
    Wr, br.reshape(1, D))


def _head_call(nums, dens, bias, batch2d, fc1_W, fc1_b, fc2_W, fc2_b, dt):
  BLK = 200
  NB = N // BLK

  def body(p_ref, d_ref, b_ref, bat_ref, w1_ref, b1_ref, w2_ref, b2_ref,
           out_ref, pool_ref):
    i = pl.program_id(0)
    den = d_ref[0] + d_ref[1] + 1e-16
    h = (p_ref[0] + p_ref[1]) / den + b_ref[...]
    bat = bat_ref[...]
    for g in range(G):
      m = jnp.max(jnp.where(bat == g, h, -1e30), axis=0, keepdims=True)

      @pl.when(i == 0)
      def _():
        pool_ref[pl.ds(g, 1), :] = m

      @pl.when(i > 0)
      def _():
        pool_ref[pl.ds(g, 1), :] = jnp.maximum(pool_ref[pl.ds(g, 1), :], m)

    @pl.when(i == NB - 1)
    def _():
      pooled = pool_ref[...]
      pooled = jnp.where(pooled > -1e29, pooled, 0.0)
      h1 = jnp.maximum(
          jnp.dot(pooled, w1_ref[...], preferred_element_type=jnp.float32)
          + b1_ref[...], 0.0)
      out_ref[...] = jnp.dot(h1, w2_ref[...],
                             preferred_element_type=jnp.float32) + b2_ref[...]

  return pl.pallas_call(
      body,
      grid=(NB,),
      in_specs=[
          pl.BlockSpec((NC, BLK, D), lambda i: (0, i, 0)),
          pl.BlockSpec((NC, BLK, 1), lambda i: (0, i, 0)),
          pl.BlockSpec((1, D), lambda i: (0, 0)),
          pl.BlockSpec((BLK, 1), lambda i: (i, 0)),
          pl.BlockSpec((D, D), lambda i: (0, 0)),
          pl.BlockSpec((1, D), lambda i: (0, 0)),
          pl.BlockSpec((D, dt), lambda i: (0, 0)),
          pl.BlockSpec((1, dt), lambda i: (0, 0)),
      ],
      out_specs=pl.BlockSpec((G, dt), lambda i: (0, 0)),
      out_shape=jax.ShapeDtypeStruct((G, dt), jnp.float32),
      scratch_shapes=[pltpu.VMEM((G, D), jnp.float32)],
  )(nums, dens, bias.reshape(1, D), batch2d, fc1_W, fc1_b.reshape(1, D),
    fc2_W, fc2_b.reshape(1, dt))


def kernel(x, edge_index, batch, Wl1, bl1, Wr1, br1, att1, bias1,
           Wl2, bl2, Wr2, br2, att2, bias2, fc1_W, fc1_b, fc2_W, fc2_b):
  src = edge_index[0]
  dst = edge_index[1]
  dt = fc2_W.shape[1]

  xl1, xr1 = _proj_call(x, Wl1, bl1, Wr1, br1)
  num1, den1 = _edge_layer_call(xl1, xr1, src, dst, att1)
  den1 = den1.reshape(NC, N, 1)

  xl2, xr2 = _combine_proj_call(num1, den1, bias1, Wl2, bl2, Wr2, br2)
  num2, den2 = _edge_layer_call(xl2, xr2, src, dst, att2)
  den2 = den2.reshape(NC, N, 1)

  return _head_call(num2, den2, bias2, batch.reshape(N, 1).astype(jnp.int32),
                    fc1_W, fc1_b, fc2_W, fc2_b, dt)


# speedup vs baseline: 1.2559x; 1.2559x over previous
import functools

import jax
import jax.numpy as jnp
from jax import lax
from jax.experimental import pallas as pl
from jax.experimental.pallas import tpu as pltpu
from jax.experimental.pallas import tpu_sc as plsc

N = 10000
E = 320000
D = 128
G = 64

NC = 2
NS = 16
NW = NC * NS
EW = E // NW
K = 80
NCH = EW // K
R0 = 624

_mesh = functools.partial(
    plsc.VectorSubcoreMesh, core_axis_name="c", subcore_axis_name="s")


def _edge_layer_call(xl, xr, src, dst, att):
  NB2 = 2
  NB4 = 4

  @functools.partial(
      pl.kernel,
      out_type=(jax.ShapeDtypeStruct((NC, N, D), jnp.float32),
                jax.ShapeDtypeStruct((NC, 1, N), jnp.float32)),
      mesh=_mesh(),
      compiler_params=pltpu.CompilerParams(needs_layout_passes=False),
      scratch_types=[
          [pltpu.VMEM((K,), jnp.int32)] * NB4,
          [pltpu.VMEM((K,), jnp.int32)] * NB4,
          [pltpu.VMEM((K, D), jnp.float32)] * NB2,
          [pltpu.VMEM((K, D), jnp.float32)] * NB2,
          [pltpu.VMEM((K + 16,), jnp.float32)] * NB2,
          pltpu.VMEM((D,), jnp.float32),
          pltpu.VMEM((256,), jnp.float32),
          pltpu.VMEM((624,), jnp.float32),
          pltpu.VMEM((8, D), jnp.float32),
          pltpu.VMEM_SHARED((N, D), jnp.float32),
          pltpu.VMEM_SHARED((N,), jnp.float32),
          [pltpu.SemaphoreType.DMA] * NB4,
          [pltpu.SemaphoreType.DMA] * NB2,
          [pltpu.SemaphoreType.DMA] * NB2,
          [pltpu.SemaphoreType.DMA] * NB2,
          [pltpu.SemaphoreType.DMA] * NB2,
      ])
  def k(xl_hbm, xr_hbm, src_hbm, dst_hbm, att_hbm, num_hbm, den_hbm,
        src_v, dst_v, gl_v, gr_v, ex_v, att_v, m_v, zden_v,
        zrow_v, acc_sh, denom_sh, sem_i, sem_g, sem_r, sem_sr, sem_se):
    cid = lax.axis_index("c")
    sid = lax.axis_index("s")
    wid = cid * NS + sid

    def zd(i, _):
      zden_v[pl.ds(i * 16, 16)] = jnp.zeros((16,), jnp.float32)
      return 0
    lax.fori_loop(0, 624 // 16, zd, 0)

    for r in range(8):
      for g in range(D // 16):
        zrow_v[r, pl.ds(g * 16, 16)] = jnp.zeros((16,), jnp.float32)

    pltpu.sync_copy(zden_v, denom_sh.at[pl.ds(sid * R0, R0)])

    def zc(i, _):
      pltpu.sync_copy(zrow_v, acc_sh.at[pl.ds(sid * R0 + i * 8, 8)])
      return 0
    lax.fori_loop(0, R0 // 8, zc, 0)

    @pl.when(sid == 0)
    def _():
      pltpu.sync_copy(zden_v.at[pl.ds(0, 16)],
                      denom_sh.at[pl.ds(NS * R0, N - NS * R0)])
      pltpu.sync_copy(zrow_v, acc_sh.at[pl.ds(NS * R0, 8)])
      pltpu.sync_copy(zrow_v, acc_sh.at[pl.ds(NS * R0 + 8, 8)])

    pltpu.sync_copy(att_hbm, att_v)
    plsc.subcore_barrier()

    lane = lax.iota(jnp.int32, 16)
    lane16 = lane * 16

    def start_idx(c, bi):
      base = wid * EW + c * K
      cps = pltpu.async_copy(src_hbm.at[pl.ds(base, K)], src_v[bi],
                             sem_i[bi])
      cpd = pltpu.async_copy(dst_hbm.at[pl.ds(base, K)], dst_v[bi],
                             sem_i[bi])
      del cps, cpd

    def wait_idx(c, bi):
      base = wid * EW + c * K
      pltpu.make_async_copy(src_hbm.at[pl.ds(base, K)], src_v[bi],
                            sem_i[bi]).wait()
      pltpu.make_async_copy(dst_hbm.at[pl.ds(base, K)], dst_v[bi],
                            sem_i[bi]).wait()

    def start_gather(b, bi):
      cpl = pltpu.async_copy(xl_hbm.at[src_v[bi]], gl_v[b], sem_g[b])
      cpr = pltpu.async_copy(xr_hbm.at[dst_v[bi]], gr_v[b], sem_r[b])
      del cpl, cpr

    def wait_gather(b, bi):
      pltpu.make_async_copy(xl_hbm.at[src_v[bi]], gl_v[b], sem_g[b]).wait()
      pltpu.make_async_copy(xr_hbm.at[dst_v[bi]], gr_v[b], sem_r[b]).wait()

    def start_scatter(b, bi):
      cpa = pltpu.async_copy(gl_v[b], acc_sh.at[dst_v[bi]], sem_sr[b],
                             add=True)
      cpe = pltpu.async_copy(ex_v[b].at[pl.ds(0, K)],
                             denom_sh.at[dst_v[bi]], sem_se[b], add=True)
      del cpa, cpe

    def wait_scatter(b, bi):
      pltpu.make_async_copy(gl_v[b], acc_sh.at[dst_v[bi]],
                            sem_sr[b]).wait()
      pltpu.make_async_copy(ex_v[b].at[pl.ds(0, K)],
                            denom_sh.at[dst_v[bi]], sem_se[b]).wait()

    def compute(b):
      def jgroup(j, _):
        jb = j * 16
        att16 = [att_v[pl.ds(g * 16, 16)] for g in range(D // 16)]
        for ei in range(16):
          e = jb + ei
          acc = jnp.zeros((16,), jnp.float32)
          for g in range(D // 16):
            s = gl_v[b][e, pl.ds(g * 16, 16)] + gr_v[b][e, pl.ds(g * 16, 16)]
            s = jnp.maximum(s, 0.2 * s)
            acc = acc + s * att16[g]
          m_v[pl.ds(ei * 16, 16)] = acc
        cols = [plsc.load_gather(m_v, [lane16 + l]) for l in range(16)]
        while len(cols) > 1:
          cols = [cols[i] + cols[i + 1] for i in range(0, len(cols), 2)]
        exv = jnp.exp(cols[0])
        ex_v[b][pl.ds(jb, 16)] = exv
        for ei in range(16):
          e = jb + ei
          a = ex_v[b][pl.ds(e, 16)][0]
          for g in range(D // 16):
            gl_v[b][e, pl.ds(g * 16, 16)] = gl_v[b][e, pl.ds(g * 16, 16)] * a
        return 0

      lax.fori_loop(0, K // 16, jgroup, 0)

    start_idx(0, 0)
    start_idx(1, 1)
    wait_idx(0, 0)
    start_gather(0, 0)

    def quad(q, _):
      for s in range(NB4):
        c = q * NB4 + s
        b = s % NB2

        @pl.when(c < NCH)
        def _():
          @pl.when(c + 2 < NCH)
          def _():
            start_idx(c + 2, (s + 2) % NB4)

          @pl.when(c + 1 < NCH)
          def _():
            @pl.when(c >= 1)
            def _():
              wait_scatter((s + 1) % NB2, (s + 1) % NB4)
            wait_idx(c + 1, (s + 1) % NB4)
            start_gather((s + 1) % NB2, (s + 1) % NB4)

          wait_gather(b, s % NB4)
          compute(b)
          start_scatter(b, s % NB4)
      return 0

    lax.fori_loop(0, (NCH + NB4 - 1) // NB4, quad, 0)

    wait_scatter((NCH - 2) % NB2, (NCH - 2) % NB4)
    wait_scatter((NCH - 1) % NB2, (NCH - 1) % NB4)

    plsc.subcore_barrier()

    pltpu.sync_copy(acc_sh.at[pl.ds(sid * R0, R0)],
                    num_hbm.at[cid, pl.ds(sid * R0, R0)])

    @pl.when(sid == 0)
    def _():
      pltpu.sync_copy(acc_sh.at[pl.ds(NS * R0, N - NS * R0)],
                      num_hbm.at[cid, pl.ds(NS * R0, N - NS * R0)])
      pltpu.sync_copy(denom_sh, den_hbm.at[cid, 0])

  return k(xl, xr, src, dst, att)


def _proj_call(x, Wl, bl, Wr, br):
  BLK = 1000

  def body(x_ref, wl_ref, bl_ref, wr_ref, br_ref, xl_ref, xr_ref):
    xb = x_ref[...]
    xl_ref[...] = jnp.dot(xb, wl_ref[...],
                          preferred_element_type=jnp.float32) + bl_ref[...]
    xr_ref[...] = jnp.dot(xb, wr_ref[...],
                          preferred_element_type=jnp.float32) + br_ref[...]

  return pl.pallas_call(
      body,
      grid=(N // BLK,),
      in_specs=[
          pl.BlockSpec((BLK, D), lambda i: (i, 0)),
          pl.BlockSpec((D, D), lambda i: (0, 0)),
          pl.BlockSpec((1, D), lambda i: (0, 0)),
          pl.BlockSpec((D, D), lambda i: (0, 0)),
          pl.BlockSpec((1, D), lambda i: (0, 0)),
      ],
      out_specs=[
          pl.BlockSpec((BLK, D), lambda i: (i, 0)),
          pl.BlockSpec((BLK, D), lambda i: (i, 0)),
      ],
      out_shape=[
          jax.ShapeDtypeStruct((N, D), jnp.float32),
          jax.ShapeDtypeStruct((N, D), jnp.float32),
      ],
  )(x, Wl, bl.reshape(1, D), Wr, br.reshape(1, D))


def _combine_proj_call(nums, dens, bias, Wl, bl, Wr, br):
  BLK = 1000

  def body(p_ref, d_ref, b_ref, wl_ref, bl_ref, wr_ref, br_ref,
           xl_ref, xr_ref):
    den = d_ref[0] + d_ref[1] + 1e-16
    h = (p_ref[0] + p_ref[1]) / den + b_ref[...]
    xl_ref[...] = jnp.dot(h, wl_ref[...],
                          preferred_element_type=jnp.float32) + bl_ref[...]
    xr_ref[...] = jnp.dot(h, wr_ref[...],
                          preferred_element_type=jnp.float32) + br_ref[...]

  return pl.pallas_call(
      body,
      grid=(N // BLK,),
      in_specs=[
          pl.BlockSpec((NC, BLK, D), lambda i: (0, i, 0)),
          pl.BlockSpec((NC, BLK, 1), lambda i: (0, i, 0)),
          pl.BlockSpec((1, D), lambda i: (0, 0)),
          pl.BlockSpec((D, D), lambda i: (0, 0)),
          pl.BlockSpec((1, D), lambda i: (0, 0)),
          pl.BlockSpec((D, D), lambda i: (0, 0)),
          pl.BlockSpec((1, D), lambda i: (0, 0)),
      ],
      out_specs=[
          pl.BlockSpec((BLK, D), lambda i: (i, 0)),
          pl.BlockSpec((BLK, D), lambda i: (i, 0)),
      ],
      out_shape=[
          jax.ShapeDtypeStruct((N, D), jnp.float32),
          jax.ShapeDtypeStruct((N, D), jnp.float32),
      ],
  )(nums, dens, bias.reshape(1, D), Wl, bl.reshape(1, D),
    Wr, br.reshape(1, D))


def _head_call(nums, dens, bias, batch2d, fc1_W, fc1_b, fc2_W, fc2_b, dt):
  BLK = 200
  NB = N // BLK

  def pool_body(p_ref, d_ref, b_ref, bat_ref, out_ref):
    den = d_ref[0] + d_ref[1] + 1e-16
    h = (p_ref[0] + p_ref[1]) / den + b_ref[...]
    bat = bat_ref[...]
    for g in range(G):
      m = jnp.where(bat == g, h, -1e30)
      out_ref[0, pl.ds(g, 1), :] = jnp.max(m, axis=0, keepdims=True)

  pooled_parts = pl.pallas_call(
      pool_body,
      grid=(NB,),
      in_specs=[
          pl.BlockSpec((NC, BLK, D), lambda i: (0, i, 0)),
          pl.BlockSpec((NC, BLK, 1), lambda i: (0, i, 0)),
          pl.BlockSpec((1, D), lambda i: (0, 0)),
          pl.BlockSpec((BLK, 1), lambda i: (i, 0)),
      ],
      out_specs=pl.BlockSpec((1, G, D), lambda i: (i, 0, 0)),
      out_shape=jax.ShapeDtypeStruct((NB, G, D), jnp.float32),
  )(nums, dens, bias.reshape(1, D), batch2d)

  def fin_body(pp_ref, w1_ref, b1_ref, w2_ref, b2_ref, out_ref):
    pooled = jnp.max(pp_ref[...], axis=0)
    pooled = jnp.where(pooled > -1e29, pooled, 0.0)
    h1 = jnp.maximum(
        jnp.dot(pooled, w1_ref[...], preferred_element_type=jnp.float32)
        + b1_ref[...], 0.0)
    out_ref[...] = jnp.dot(h1, w2_ref[...],
                           preferred_element_type=jnp.float32) + b2_ref[...]

  return pl.pallas_call(
      fin_body,
      out_shape=jax.ShapeDtypeStruct((G, dt), jnp.float32),
  )(pooled_parts, fc1_W, fc1_b.reshape(1, D), fc2_W, fc2_b.reshape(1, dt))


def kernel(x, edge_index, batch, Wl1, bl1, Wr1, br1, att1, bias1,
           Wl2, bl2, Wr2, br2, att2, bias2, fc1_W, fc1_b, fc2_W, fc2_b):
  src = edge_index[0]
  dst = edge_index[1]
  dt = fc2_W.shape[1]

  xl1, xr1 = _proj_call(x, Wl1, bl1, Wr1, br1)
  num1, den1 = _edge_layer_call(xl1, xr1, src, dst, att1)
  den1 = den1.reshape(NC, N, 1)

  xl2, xr2 = _combine_proj_call(num1, den1, bias1, Wl2, bl2, Wr2, br2)
  num2, den2 = _edge_layer_call(xl2, xr2, src, dst, att2)
  den2 = den2.reshape(NC, N, 1)

  return _head_call(num2, den2, bias2, batch.reshape(N, 1).astype(jnp.int32),
                    fc1_W, fc1_b, fc2_W, fc2_b, dt)


# speedup vs baseline: 1.2609x; 1.0040x over previous
import functools

import jax
import jax.numpy as jnp
from jax import lax
from jax.experimental import pallas as pl
from jax.experimental.pallas import tpu as pltpu
from jax.experimental.pallas import tpu_sc as plsc

N = 10000
E = 320000
D = 128
G = 64

NC = 2
NS = 16
NW = NC * NS
EW = E // NW
K = 80
NCH = EW // K
R0 = 624

_mesh = functools.partial(
    plsc.VectorSubcoreMesh, core_axis_name="c", subcore_axis_name="s")


def _edge_layer_call(xl, xr, src, dst, att):
  NB2 = 2
  NB4 = 4

  @functools.partial(
      pl.kernel,
      out_type=(jax.ShapeDtypeStruct((NC, N, D), jnp.float32),
                jax.ShapeDtypeStruct((NC, 1, N), jnp.float32)),
      mesh=_mesh(),
      compiler_params=pltpu.CompilerParams(needs_layout_passes=False),
      scratch_types=[
          [pltpu.VMEM((K,), jnp.int32)] * NB4,
          [pltpu.VMEM((K,), jnp.int32)] * NB4,
          [pltpu.VMEM((K, D), jnp.float32)] * NB2,
          [pltpu.VMEM((K, D), jnp.float32)] * NB2,
          [pltpu.VMEM((K + 16,), jnp.float32)] * NB2,
          pltpu.VMEM((D,), jnp.float32),
          pltpu.VMEM((256,), jnp.float32),
          pltpu.VMEM((624,), jnp.float32),
          pltpu.VMEM((8, D), jnp.float32),
          pltpu.VMEM_SHARED((N, D), jnp.float32),
          pltpu.VMEM_SHARED((N,), jnp.float32),
          [pltpu.SemaphoreType.DMA] * NB4,
          [pltpu.SemaphoreType.DMA] * NB2,
          [pltpu.SemaphoreType.DMA] * NB2,
          [pltpu.SemaphoreType.DMA] * NB2,
          [pltpu.SemaphoreType.DMA] * NB2,
      ])
  def k(xl_hbm, xr_hbm, src_hbm, dst_hbm, att_hbm, num_hbm, den_hbm,
        src_v, dst_v, gl_v, gr_v, ex_v, att_v, m_v, zden_v,
        zrow_v, acc_sh, denom_sh, sem_i, sem_g, sem_r, sem_sr, sem_se):
    cid = lax.axis_index("c")
    sid = lax.axis_index("s")
    wid = cid * NS + sid

    def zd(i, _):
      zden_v[pl.ds(i * 16, 16)] = jnp.zeros((16,), jnp.float32)
      return 0
    lax.fori_loop(0, 624 // 16, zd, 0)

    for r in range(8):
      for g in range(D // 16):
        zrow_v[r, pl.ds(g * 16, 16)] = jnp.zeros((16,), jnp.float32)

    pltpu.sync_copy(zden_v, denom_sh.at[pl.ds(sid * R0, R0)])

    def zc(i, _):
      pltpu.sync_copy(zrow_v, acc_sh.at[pl.ds(sid * R0 + i * 8, 8)])
      return 0
    lax.fori_loop(0, R0 // 8, zc, 0)

    @pl.when(sid == 0)
    def _():
      pltpu.sync_copy(zden_v.at[pl.ds(0, 16)],
                      denom_sh.at[pl.ds(NS * R0, N - NS * R0)])
      pltpu.sync_copy(zrow_v, acc_sh.at[pl.ds(NS * R0, 8)])
      pltpu.sync_copy(zrow_v, acc_sh.at[pl.ds(NS * R0 + 8, 8)])

    pltpu.sync_copy(att_hbm, att_v)
    plsc.subcore_barrier()

    lane = lax.iota(jnp.int32, 16)
    lane16 = lane * 16

    def start_idx(c, bi):
      base = wid * EW + c * K
      cps = pltpu.async_copy(src_hbm.at[pl.ds(base, K)], src_v[bi],
                             sem_i[bi])
      cpd = pltpu.async_copy(dst_hbm.at[pl.ds(base, K)], dst_v[bi],
                             sem_i[bi])
      del cps, cpd

    def wait_idx(c, bi):
      base = wid * EW + c * K
      pltpu.make_async_copy(src_hbm.at[pl.ds(base, K)], src_v[bi],
                            sem_i[bi]).wait()
      pltpu.make_async_copy(dst_hbm.at[pl.ds(base, K)], dst_v[bi],
                            sem_i[bi]).wait()

    def start_gather_l(b, bi):
      cpl = pltpu.async_copy(xl_hbm.at[src_v[bi]], gl_v[b], sem_g[b])
      del cpl

    def start_gather_r(b, bi):
      cpr = pltpu.async_copy(xr_hbm.at[dst_v[bi]], gr_v[b], sem_r[b])
      del cpr

    def wait_gather(b, bi):
      pltpu.make_async_copy(xl_hbm.at[src_v[bi]], gl_v[b], sem_g[b]).wait()
      pltpu.make_async_copy(xr_hbm.at[dst_v[bi]], gr_v[b], sem_r[b]).wait()

    def start_scatter(b, bi):
      cpa = pltpu.async_copy(gl_v[b], acc_sh.at[dst_v[bi]], sem_sr[b],
                             add=True)
      cpe = pltpu.async_copy(ex_v[b].at[pl.ds(0, K)],
                             denom_sh.at[dst_v[bi]], sem_se[b], add=True)
      del cpa, cpe

    def wait_scatter_rows(b, bi):
      pltpu.make_async_copy(gl_v[b], acc_sh.at[dst_v[bi]],
                            sem_sr[b]).wait()

    def wait_scatter_ex(b, bi):
      pltpu.make_async_copy(ex_v[b].at[pl.ds(0, K)],
                            denom_sh.at[dst_v[bi]], sem_se[b]).wait()

    def wait_scatter(b, bi):
      wait_scatter_rows(b, bi)
      wait_scatter_ex(b, bi)

    def compute(b):
      def jgroup(j, _):
        jb = j * 16
        att16 = [att_v[pl.ds(g * 16, 16)] for g in range(D // 16)]
        for ei in range(16):
          e = jb + ei
          acc = jnp.zeros((16,), jnp.float32)
          for g in range(D // 16):
            s = gl_v[b][e, pl.ds(g * 16, 16)] + gr_v[b][e, pl.ds(g * 16, 16)]
            s = jnp.maximum(s, 0.2 * s)
            acc = acc + s * att16[g]
          m_v[pl.ds(ei * 16, 16)] = acc
        cols = [plsc.load_gather(m_v, [lane16 + l]) for l in range(16)]
        while len(cols) > 1:
          cols = [cols[i] + cols[i + 1] for i in range(0, len(cols), 2)]
        exv = jnp.exp(cols[0])
        ex_v[b][pl.ds(jb, 16)] = exv
        for ei in range(16):
          e = jb + ei
          a = ex_v[b][pl.ds(e, 16)][0]
          for g in range(D // 16):
            gl_v[b][e, pl.ds(g * 16, 16)] = gl_v[b][e, pl.ds(g * 16, 16)] * a
        return 0

      lax.fori_loop(0, K // 16, jgroup, 0)

    start_idx(0, 0)
    start_idx(1, 1)
    wait_idx(0, 0)
    start_gather_r(0, 0)
    start_gather_l(0, 0)

    def quad(q, _):
      for s in range(NB4):
        c = q * NB4 + s
        b = s % NB2

        @pl.when(c < NCH)
        def _():
          @pl.when(c + 2 < NCH)
          def _():
            start_idx(c + 2, (s + 2) % NB4)

          @pl.when(c + 1 < NCH)
          def _():
            wait_idx(c + 1, (s + 1) % NB4)
            start_gather_r((s + 1) % NB2, (s + 1) % NB4)

            @pl.when(c >= 1)
            def _():
              wait_scatter_rows((s + 1) % NB2, (s + 1) % NB4)
            start_gather_l((s + 1) % NB2, (s + 1) % NB4)

          wait_gather(b, s % NB4)

          @pl.when(c >= 2)
          def _():
            wait_scatter_ex(b, (s + 2) % NB4)
          compute(b)
          start_scatter(b, s % NB4)
      return 0

    lax.fori_loop(0, (NCH + NB4 - 1) // NB4, quad, 0)

    wait_scatter((NCH - 2) % NB2, (NCH - 2) % NB4)
    wait_scatter((NCH - 1) % NB2, (NCH - 1) % NB4)

    plsc.subcore_barrier()

    pltpu.sync_copy(acc_sh.at[pl.ds(sid * R0, R0)],
                    num_hbm.at[cid, pl.ds(sid * R0, R0)])

    @pl.when(sid == 0)
    def _():
      pltpu.sync_copy(acc_sh.at[pl.ds(NS * R0, N - NS * R0)],
                      num_hbm.at[cid, pl.ds(NS * R0, N - NS * R0)])
      pltpu.sync_copy(denom_sh, den_hbm.at[cid, 0])

  return k(xl, xr, src, dst, att)


def _proj_call(x, Wl, bl, Wr, br):
  BLK = 1000

  def body(x_ref, wl_ref, bl_ref, wr_ref, br_ref, xl_ref, xr_ref):
    xb = x_ref[...]
    xl_ref[...] = jnp.dot(xb, wl_ref[...],
                          preferred_element_type=jnp.float32) + bl_ref[...]
    xr_ref[...] = jnp.dot(xb, wr_ref[...],
                          preferred_element_type=jnp.float32) + br_ref[...]

  return pl.pallas_call(
      body,
      grid=(N // BLK,),
      in_specs=[
          pl.BlockSpec((BLK, D), lambda i: (i, 0)),
          pl.BlockSpec((D, D), lambda i: (0, 0)),
          pl.BlockSpec((1, D), lambda i: (0, 0)),
          pl.BlockSpec((D, D), lambda i: (0, 0)),
          pl.BlockSpec((1, D), lambda i: (0, 0)),
      ],
      out_specs=[
          pl.BlockSpec((BLK, D), lambda i: (i, 0)),
          pl.BlockSpec((BLK, D), lambda i: (i, 0)),
      ],
      out_shape=[
          jax.ShapeDtypeStruct((N, D), jnp.float32),
          jax.ShapeDtypeStruct((N, D), jnp.float32),
      ],
  )(x, Wl, bl.reshape(1, D), Wr, br.reshape(1, D))


def _combine_proj_call(nums, dens, bias, Wl, bl, Wr, br):
  BLK = 1000

  def body(p_ref, d_ref, b_ref, wl_ref, bl_ref, wr_ref, br_ref,
           xl_ref, xr_ref):
    den = d_ref[0] + d_ref[1] + 1e-16
    h = (p_ref[0] + p_ref[1]) / den + b_ref[...]
    xl_ref[...] = jnp.dot(h, wl_ref[...],
                          preferred_element_type=jnp.float32) + bl_ref[...]
    xr_ref[...] = jnp.dot(h, wr_ref[...],
                          preferred_element_type=jnp.float32) + br_ref[...]

  return pl.pallas_call(
      body,
      grid=(N // BLK,),
      in_specs=[
          pl.BlockSpec((NC, BLK, D), lambda i: (0, i, 0)),
          pl.BlockSpec((NC, BLK, 1), lambda i: (0, i, 0)),
          pl.BlockSpec((1, D), lambda i: (0, 0)),
          pl.BlockSpec((D, D), lambda i: (0, 0)),
          pl.BlockSpec((1, D), lambda i: (0, 0)),
          pl.BlockSpec((D, D), lambda i: (0, 0)),
          pl.BlockSpec((1, D), lambda i: (0, 0)),
      ],
      out_specs=[
          pl.BlockSpec((BLK, D), lambda i: (i, 0)),
          pl.BlockSpec((BLK, D), lambda i: (i, 0)),
      ],
      out_shape=[
          jax.ShapeDtypeStruct((N, D), jnp.float32),
          jax.ShapeDtypeStruct((N, D), jnp.float32),
      ],
  )(nums, dens, bias.reshape(1, D), Wl, bl.reshape(1, D),
    Wr, br.reshape(1, D))


def _head_call(nums, dens, bias, batch2d, fc1_W, fc1_b, fc2_W, fc2_b, dt):
  BLK = 200
  NB = N // BLK

  def pool_body(p_ref, d_ref, b_ref, bat_ref, out_ref):
    den = d_ref[0] + d_ref[1] + 1e-16
    h = (p_ref[0] + p_ref[1]) / den + b_ref[...]
    bat = bat_ref[...]
    for g in range(G):
      m = jnp.where(bat == g, h, -1e30)
      out_ref[0, pl.ds(g, 1), :] = jnp.max(m, axis=0, keepdims=True)

  pooled_parts = pl.pallas_call(
      pool_body,
      grid=(NB,),
      in_specs=[
          pl.BlockSpec((NC, BLK, D), lambda i: (0, i, 0)),
          pl.BlockSpec((NC, BLK, 1), lambda i: (0, i, 0)),
          pl.BlockSpec((1, D), lambda i: (0, 0)),
          pl.BlockSpec((BLK, 1), lambda i: (i, 0)),
      ],
      out_specs=pl.BlockSpec((1, G, D), lambda i: (i, 0, 0)),
      out_shape=jax.ShapeDtypeStruct((NB, G, D), jnp.float32),
  )(nums, dens, bias.reshape(1, D), batch2d)

  def fin_body(pp_ref, w1_ref, b1_ref, w2_ref, b2_ref, out_ref):
    pooled = jnp.max(pp_ref[...], axis=0)
    pooled = jnp.where(pooled > -1e29, pooled, 0.0)
    h1 = jnp.maximum(
        jnp.dot(pooled, w1_ref[...], preferred_element_type=jnp.float32)
        + b1_ref[...], 0.0)
    out_ref[...] = jnp.dot(h1, w2_ref[...],
                           preferred_element_type=jnp.float32) + b2_ref[...]

  return pl.pallas_call(
      fin_body,
      out_shape=jax.ShapeDtypeStruct((G, dt), jnp.float32),
  )(pooled_parts, fc1_W, fc1_b.reshape(1, D), fc2_W, fc2_b.reshape(1, dt))


def kernel(x, edge_index, batch, Wl1, bl1, Wr1, br1, att1, bias1,
           Wl2, bl2, Wr2, br2, att2, bias2, fc1_W, fc1_b, fc2_W, fc2_b):
  src = edge_index[0]
  dst = edge_index[1]
  dt = fc2_W.shape[1]

  xl1, xr1 = _proj_call(x, Wl1, bl1, Wr1, br1)
  num1, den1 = _edge_layer_call(xl1, xr1, src, dst, att1)
  den1 = den1.reshape(NC, N, 1)

  xl2, xr2 = _combine_proj_call(num1, den1, bias1, Wl2, bl2, Wr2, br2)
  num2, den2 = _edge_layer_call(xl2, xr2, src, dst, att2)
  den2 = den2.reshape(NC, N, 1)

  return _head_call(num2, den2, bias2, batch.reshape(N, 1).astype(jnp.int32),
                    fc1_W, fc1_b, fc2_W, fc2_b, dt)
